# baseline (device time: 191757 ns/iter reference)
import jax
import jax.numpy as jnp
from jax import lax
from jax.experimental import pallas as pl
from jax.experimental.pallas import tpu as pltpu

N_DEV = 4
SQ = 256
SKV = 4096
HQ = 8
DH = 128
D = HQ * DH
BLK = 64
SCALE = 0.08838834764831843
NEG = -1e9


def kernel(x, Wq, K_ext, V_ext, Wo):
    xb = x[0].astype(jnp.bfloat16)
    wq = Wq.astype(jnp.bfloat16)
    kb = K_ext[0].reshape(SKV, D).astype(jnp.bfloat16)
    vb = V_ext[0].reshape(SKV, D).astype(jnp.bfloat16)
    wo = Wo.astype(jnp.bfloat16)

    def body(x_ref, wq_ref, k_ref, v_ref, wo_ref, out_ref,
             qbuf, accbuf, mlbuf, ctxbuf, send_sems, recv_sems):
        p = lax.axis_index("i")
        left = lax.rem(p - 1 + N_DEV, N_DEV)
        right = lax.rem(p + 1, N_DEV)

        barrier = pltpu.get_barrier_semaphore()
        for nbr in (left, right):
            pl.semaphore_signal(barrier, inc=1, device_id=(nbr,),
                                device_id_type=pl.DeviceIdType.MESH)
        pl.semaphore_wait(barrier, 2)

        qbuf[0] = jnp.dot(x_ref[...], wq_ref[...],
                          preferred_element_type=jnp.float32).astype(jnp.bfloat16)

        colb = p * (SKV // BLK) + lax.broadcasted_iota(jnp.int32, (SQ, SKV), 1) // BLK

        for h in range(N_DEV):
            o = lax.rem(p - h + N_DEV, N_DEV)
            rowb = o * (SQ // BLK) + lax.broadcasted_iota(jnp.int32, (SQ, SKV), 0) // BLK
            mask = (rowb == colb) | (colb == 0) | ((rowb + colb) % 3 == 0)
            for hd in range(HQ):
                sl = pl.ds(hd * DH, DH)
                qh = qbuf[h, :, sl]
                s = lax.dot_general(qh, k_ref[:, sl],
                                    (((1,), (1,)), ((), ())),
                                    preferred_element_type=jnp.float32) * SCALE
                s = jnp.where(mask, s, NEG)
                smax = jnp.max(s, axis=1, keepdims=True)
                if h == 0:
                    m_new = smax
                    pe = jnp.exp(s - m_new)
                    l_new = jnp.sum(pe, axis=1, keepdims=True)
                    acc = lax.dot_general(pe.astype(jnp.bfloat16), v_ref[:, sl],
                                          (((1,), (0,)), ((), ())),
                                          preferred_element_type=jnp.float32)
                else:
                    m_old = mlbuf[h, :, hd:hd + 1]
                    l_old = mlbuf[h, :, HQ + hd:HQ + hd + 1]
                    m_new = jnp.maximum(m_old, smax)
                    alpha = jnp.exp(m_old - m_new)
                    pe = jnp.exp(s - m_new)
                    l_new = l_old * alpha + jnp.sum(pe, axis=1, keepdims=True)
                    acc = accbuf[h, :, sl] * alpha + lax.dot_general(
                        pe.astype(jnp.bfloat16), v_ref[:, sl],
                        (((1,), (0,)), ((), ())),
                        preferred_element_type=jnp.float32)
                accbuf[h, :, sl] = acc
                mlbuf[h, :, hd:hd + 1] = m_new
                mlbuf[h, :, HQ + hd:HQ + hd + 1] = l_new

            rdmas = []
            for j, buf in enumerate((qbuf, accbuf, mlbuf)):
                r = pltpu.make_async_remote_copy(
                    src_ref=buf.at[h], dst_ref=buf.at[h + 1],
                    send_sem=send_sems.at[h, j], recv_sem=recv_sems.at[h, j],
                    device_id=(right,), device_id_type=pl.DeviceIdType.MESH)
                r.start()
                rdmas.append(r)
            for r in rdmas:
                r.wait()

        for hd in range(HQ):
            sl = pl.ds(hd * DH, DH)
            ctxbuf[:, sl] = (accbuf[N_DEV, :, sl] /
                             mlbuf[N_DEV, :, HQ + hd:HQ + hd + 1]).astype(jnp.bfloat16)
        out_ref[...] = lax.dot_general(ctxbuf[...], wo_ref[...],
                                       (((1,), (0,)), ((), ())),
                                       preferred_element_type=jnp.float32)

    out = pl.pallas_call(
        body,
        out_shape=jax.ShapeDtypeStruct((SQ, D), jnp.float32),
        in_specs=[pl.BlockSpec(memory_space=pltpu.VMEM)] * 5,
        out_specs=pl.BlockSpec(memory_space=pltpu.VMEM),
        scratch_shapes=[
            pltpu.VMEM((N_DEV + 1, SQ, D), jnp.bfloat16),
            pltpu.VMEM((N_DEV + 1, SQ, D), jnp.float32),
            pltpu.VMEM((N_DEV + 1, SQ, 2 * HQ), jnp.float32),
            pltpu.VMEM((SQ, D), jnp.bfloat16),
            pltpu.SemaphoreType.DMA((N_DEV, 3)),
            pltpu.SemaphoreType.DMA((N_DEV, 3)),
        ],
        compiler_params=pltpu.CompilerParams(collective_id=0),
    )(xb, wq, kb, vb, wo)
    return out[None]


# device time: 118578 ns/iter; 1.6171x vs baseline; 1.6171x over previous
import jax
import jax.numpy as jnp
from jax import lax
from jax.experimental import pallas as pl
from jax.experimental.pallas import tpu as pltpu

N_DEV = 4
SQ = 256
SKV = 4096
HQ = 8
DH = 128
D = HQ * DH
BLK = 64
SCALE = 0.08838834764831843
NEG = -1e9


def kernel(x, Wq, K_ext, V_ext, Wo):
    xb = x[0].astype(jnp.bfloat16)
    wq = Wq.astype(jnp.bfloat16)
    kb = K_ext[0].reshape(SKV, D).astype(jnp.bfloat16)
    vb = V_ext[0].reshape(SKV, D).astype(jnp.bfloat16)
    wo = Wo.astype(jnp.bfloat16)

    def body(x_ref, wq_ref, k_ref, v_ref, wo_ref, out_ref,
             qbuf, accbuf, mlbuf, cbuf, mlloc, ctxbuf,
             qsend_sems, qrecv_sems, send_sems, recv_sems):
        p = lax.axis_index("i")
        left = lax.rem(p - 1 + N_DEV, N_DEV)
        right = lax.rem(p + 1, N_DEV)

        barrier = pltpu.get_barrier_semaphore()
        for nbr in (left, right):
            pl.semaphore_signal(barrier, inc=1, device_id=(nbr,),
                                device_id_type=pl.DeviceIdType.MESH)
        pl.semaphore_wait(barrier, 2)

        qbuf[0] = jnp.dot(x_ref[...], wq_ref[...],
                          preferred_element_type=jnp.float32).astype(jnp.bfloat16)

        colb = p * (SKV // BLK) + lax.broadcasted_iota(jnp.int32, (SQ, SKV), 1) // BLK

        pending = []

        for h in range(N_DEV):
            o = lax.rem(p - h + N_DEV, N_DEV)

            if h > 0:
                qr = pltpu.make_async_remote_copy(
                    src_ref=qbuf.at[h], dst_ref=qbuf.at[h],
                    send_sem=qsend_sems.at[h - 1], recv_sem=qrecv_sems.at[h - 1],
                    device_id=(left,), device_id_type=pl.DeviceIdType.MESH)
                qr.wait_recv()
            if h < N_DEV - 1:
                qs = pltpu.make_async_remote_copy(
                    src_ref=qbuf.at[h], dst_ref=qbuf.at[h + 1],
                    send_sem=qsend_sems.at[h], recv_sem=qrecv_sems.at[h],
                    device_id=(right,), device_id_type=pl.DeviceIdType.MESH)
                qs.start()
                pending.append(qs)

            rowb = o * (SQ // BLK) + lax.broadcasted_iota(jnp.int32, (SQ, SKV), 0) // BLK
            mask = (rowb == colb) | (colb == 0) | ((rowb + colb) % 3 == 0)
            for hd in range(HQ):
                sl = pl.ds(hd * DH, DH)
                qh = qbuf[h, :, sl]
                s = lax.dot_general(qh, k_ref[:, sl],
                                    (((1,), (1,)), ((), ())),
                                    preferred_element_type=jnp.float32) * SCALE
                s = jnp.where(mask, s, NEG)
                m_loc = jnp.max(s, axis=1, keepdims=True)
                pe = jnp.exp(s - m_loc)
                l_loc = jnp.sum(pe, axis=1, keepdims=True)
                cbuf[:, sl] = lax.dot_general(pe.astype(jnp.bfloat16), v_ref[:, sl],
                                              (((1,), (0,)), ((), ())),
                                              preferred_element_type=jnp.float32)
                mlloc[:, hd:hd + 1] = m_loc
                mlloc[:, HQ + hd:HQ + hd + 1] = l_loc

            if h == 0:
                accbuf[0] = cbuf[...].astype(jnp.bfloat16)
                mlbuf[0] = mlloc[...]
            else:
                ar = pltpu.make_async_remote_copy(
                    src_ref=accbuf.at[h], dst_ref=accbuf.at[h],
                    send_sem=send_sems.at[h - 1, 0], recv_sem=recv_sems.at[h - 1, 0],
                    device_id=(left,), device_id_type=pl.DeviceIdType.MESH)
                ar.wait_recv()
                mr = pltpu.make_async_remote_copy(
                    src_ref=mlbuf.at[h], dst_ref=mlbuf.at[h],
                    send_sem=send_sems.at[h - 1, 1], recv_sem=recv_sems.at[h - 1, 1],
                    device_id=(left,), device_id_type=pl.DeviceIdType.MESH)
                mr.wait_recv()
                for hd in range(HQ):
                    sl = pl.ds(hd * DH, DH)
                    m_in = mlbuf[h, :, hd:hd + 1]
                    l_in = mlbuf[h, :, HQ + hd:HQ + hd + 1]
                    m_loc = mlloc[:, hd:hd + 1]
                    l_loc = mlloc[:, HQ + hd:HQ + hd + 1]
                    m_new = jnp.maximum(m_in, m_loc)
                    a_in = jnp.exp(m_in - m_new)
                    a_loc = jnp.exp(m_loc - m_new)
                    acc = (accbuf[h, :, sl].astype(jnp.float32) * a_in
                           + cbuf[:, sl] * a_loc)
                    accbuf[h, :, sl] = acc.astype(jnp.bfloat16)
                    mlbuf[h, :, hd:hd + 1] = m_new
                    mlbuf[h, :, HQ + hd:HQ + hd + 1] = l_in * a_in + l_loc * a_loc

            for j, buf in enumerate((accbuf, mlbuf)):
                r = pltpu.make_async_remote_copy(
                    src_ref=buf.at[h], dst_ref=buf.at[h + 1],
                    send_sem=send_sems.at[h, j], recv_sem=recv_sems.at[h, j],
                    device_id=(right,), device_id_type=pl.DeviceIdType.MESH)
                r.start()
                pending.append(r)

        for j, buf in enumerate((accbuf, mlbuf)):
            rr = pltpu.make_async_remote_copy(
                src_ref=buf.at[N_DEV], dst_ref=buf.at[N_DEV],
                send_sem=send_sems.at[N_DEV - 1, j],
                recv_sem=recv_sems.at[N_DEV - 1, j],
                device_id=(left,), device_id_type=pl.DeviceIdType.MESH)
            rr.wait_recv()

        for hd in range(HQ):
            sl = pl.ds(hd * DH, DH)
            ctxbuf[:, sl] = (accbuf[N_DEV, :, sl].astype(jnp.float32) /
                             mlbuf[N_DEV, :, HQ + hd:HQ + hd + 1]).astype(jnp.bfloat16)
        out_ref[...] = lax.dot_general(ctxbuf[...], wo_ref[...],
                                       (((1,), (0,)), ((), ())),
                                       preferred_element_type=jnp.float32)

        for r in pending:
            r.wait_send()

    out = pl.pallas_call(
        body,
        out_shape=jax.ShapeDtypeStruct((SQ, D), jnp.float32),
        in_specs=[pl.BlockSpec(memory_space=pltpu.VMEM)] * 5,
        out_specs=pl.BlockSpec(memory_space=pltpu.VMEM),
        scratch_shapes=[
            pltpu.VMEM((N_DEV, SQ, D), jnp.bfloat16),
            pltpu.VMEM((N_DEV + 1, SQ, D), jnp.bfloat16),
            pltpu.VMEM((N_DEV + 1, SQ, 2 * HQ), jnp.float32),
            pltpu.VMEM((SQ, D), jnp.float32),
            pltpu.VMEM((SQ, 2 * HQ), jnp.float32),
            pltpu.VMEM((SQ, D), jnp.bfloat16),
            pltpu.SemaphoreType.DMA((N_DEV - 1,)),
            pltpu.SemaphoreType.DMA((N_DEV - 1,)),
            pltpu.SemaphoreType.DMA((N_DEV, 2)),
            pltpu.SemaphoreType.DMA((N_DEV, 2)),
        ],
        compiler_params=pltpu.CompilerParams(collective_id=0),
    )(xb, wq, kb, vb, wo)
    return out[None]


# device time: 101789 ns/iter; 1.8839x vs baseline; 1.1649x over previous
import jax
import jax.numpy as jnp
from jax import lax
from jax.experimental import pallas as pl
from jax.experimental.pallas import tpu as pltpu

N_DEV = 4
SQ = 256
SKV = 4096
HQ = 8
DH = 128
D = HQ * DH
BLK = 64
SCALE = 0.08838834764831843
NEG = -1e9


def kernel(x, Wq, K_ext, V_ext, Wo):
    xb = x[0].astype(jnp.bfloat16)
    wq = Wq.astype(jnp.bfloat16)
    kb = K_ext[0].reshape(SKV, D).astype(jnp.bfloat16)
    vb = V_ext[0].reshape(SKV, D).astype(jnp.bfloat16)
    wo = Wo.astype(jnp.bfloat16)

    def body(x_ref, wq_ref, k_ref, v_ref, wo_ref, out_ref,
             qbuf, accbuf, lbuf, cbuf, lloc, ctxbuf,
             qsend_sems, qrecv_sems, send_sems, recv_sems):
        p = lax.axis_index("i")
        left = lax.rem(p - 1 + N_DEV, N_DEV)
        right = lax.rem(p + 1, N_DEV)

        barrier = pltpu.get_barrier_semaphore()
        for nbr in (left, right):
            pl.semaphore_signal(barrier, inc=1, device_id=(nbr,),
                                device_id_type=pl.DeviceIdType.MESH)
        pl.semaphore_wait(barrier, 2)

        qbuf[0] = (jnp.dot(x_ref[...], wq_ref[...],
                           preferred_element_type=jnp.float32)
                   * SCALE).astype(jnp.bfloat16)

        colb = p * (SKV // BLK) + lax.broadcasted_iota(jnp.int32, (SQ, SKV), 1) // BLK

        pending = []

        for h in range(N_DEV):
            o = lax.rem(p - h + N_DEV, N_DEV)

            if h > 0:
                qr = pltpu.make_async_remote_copy(
                    src_ref=qbuf.at[h], dst_ref=qbuf.at[h],
                    send_sem=qsend_sems.at[h - 1], recv_sem=qrecv_sems.at[h - 1],
                    device_id=(left,), device_id_type=pl.DeviceIdType.MESH)
                qr.wait_recv()
            if h < N_DEV - 1:
                qs = pltpu.make_async_remote_copy(
                    src_ref=qbuf.at[h], dst_ref=qbuf.at[h + 1],
                    send_sem=qsend_sems.at[h], recv_sem=qrecv_sems.at[h],
                    device_id=(right,), device_id_type=pl.DeviceIdType.MESH)
                qs.start()
                pending.append(qs)

            rowb = o * (SQ // BLK) + lax.broadcasted_iota(jnp.int32, (SQ, SKV), 0) // BLK
            mask = (rowb == colb) | (colb == 0) | ((rowb + colb) % 3 == 0)
            for hd in range(HQ):
                sl = pl.ds(hd * DH, DH)
                qh = qbuf[h, :, sl]
                s = lax.dot_general(qh, k_ref[:, sl],
                                    (((1,), (1,)), ((), ())),
                                    preferred_element_type=jnp.float32)
                pe = jnp.exp(jnp.where(mask, s, NEG))
                lloc[:, hd:hd + 1] = jnp.sum(pe, axis=1, keepdims=True)
                cbuf[:, sl] = lax.dot_general(pe.astype(jnp.bfloat16), v_ref[:, sl],
                                              (((1,), (0,)), ((), ())),
                                              preferred_element_type=jnp.float32)

            if h == 0:
                accbuf[0] = cbuf[...].astype(jnp.bfloat16)
                lbuf[0] = lloc[...]
            else:
                ar = pltpu.make_async_remote_copy(
                    src_ref=accbuf.at[h], dst_ref=accbuf.at[h],
                    send_sem=send_sems.at[h - 1, 0], recv_sem=recv_sems.at[h - 1, 0],
                    device_id=(left,), device_id_type=pl.DeviceIdType.MESH)
                ar.wait_recv()
                lr = pltpu.make_async_remote_copy(
                    src_ref=lbuf.at[h], dst_ref=lbuf.at[h],
                    send_sem=send_sems.at[h - 1, 1], recv_sem=recv_sems.at[h - 1, 1],
                    device_id=(left,), device_id_type=pl.DeviceIdType.MESH)
                lr.wait_recv()
                accbuf[h] = (accbuf[h].astype(jnp.float32)
                             + cbuf[...]).astype(jnp.bfloat16)
                lbuf[h] = lbuf[h] + lloc[...]

            for j, buf in enumerate((accbuf, lbuf)):
                r = pltpu.make_async_remote_copy(
                    src_ref=buf.at[h], dst_ref=buf.at[h + 1],
                    send_sem=send_sems.at[h, j], recv_sem=recv_sems.at[h, j],
                    device_id=(right,), device_id_type=pl.DeviceIdType.MESH)
                r.start()
                pending.append(r)

        for j, buf in enumerate((accbuf, lbuf)):
            rr = pltpu.make_async_remote_copy(
                src_ref=buf.at[N_DEV], dst_ref=buf.at[N_DEV],
                send_sem=send_sems.at[N_DEV - 1, j],
                recv_sem=recv_sems.at[N_DEV - 1, j],
                device_id=(left,), device_id_type=pl.DeviceIdType.MESH)
            rr.wait_recv()

        for hd in range(HQ):
            sl = pl.ds(hd * DH, DH)
            ctxbuf[:, sl] = (accbuf[N_DEV, :, sl].astype(jnp.float32) /
                             lbuf[N_DEV, :, hd:hd + 1]).astype(jnp.bfloat16)
        out_ref[...] = lax.dot_general(ctxbuf[...], wo_ref[...],
                                       (((1,), (0,)), ((), ())),
                                       preferred_element_type=jnp.float32)

        for r in pending:
            r.wait_send()

    out = pl.pallas_call(
        body,
        out_shape=jax.ShapeDtypeStruct((SQ, D), jnp.float32),
        in_specs=[pl.BlockSpec(memory_space=pltpu.VMEM)] * 5,
        out_specs=pl.BlockSpec(memory_space=pltpu.VMEM),
        scratch_shapes=[
            pltpu.VMEM((N_DEV, SQ, D), jnp.bfloat16),
            pltpu.VMEM((N_DEV + 1, SQ, D), jnp.bfloat16),
            pltpu.VMEM((N_DEV + 1, SQ, HQ), jnp.float32),
            pltpu.VMEM((SQ, D), jnp.float32),
            pltpu.VMEM((SQ, HQ), jnp.float32),
            pltpu.VMEM((SQ, D), jnp.bfloat16),
            pltpu.SemaphoreType.DMA((N_DEV - 1,)),
            pltpu.SemaphoreType.DMA((N_DEV - 1,)),
            pltpu.SemaphoreType.DMA((N_DEV, 2)),
            pltpu.SemaphoreType.DMA((N_DEV, 2)),
        ],
        compiler_params=pltpu.CompilerParams(collective_id=0),
    )(xb, wq, kb, vb, wo)
    return out[None]
